# baseline (speedup 1.0000x reference)
import functools

import jax
import jax.numpy as jnp
from jax import lax
from jax.experimental import pallas as pl
from jax.experimental.pallas import tpu as pltpu
from jax.experimental.pallas import tpu_sc as plsc

_NC = 2
_NS = 16
_NW = _NC * _NS
_CHUNK = 128

_EPS = 1e-5


def _make_gather(batch, dim2):
    b_per_w = batch // _NW
    n_chunks = b_per_w // _CHUNK
    assert b_per_w * _NW == batch and n_chunks * _CHUNK == b_per_w

    mesh = plsc.VectorSubcoreMesh(core_axis_name="c", subcore_axis_name="s")

    @functools.partial(
        pl.kernel,
        mesh=mesh,
        out_type=jax.ShapeDtypeStruct((batch, dim2), jnp.float32),
        scratch_types=[
            pltpu.VMEM((n_chunks, _CHUNK), jnp.int32),
            pltpu.VMEM((b_per_w, dim2), jnp.float32),
            pltpu.SemaphoreType.DMA,
        ],
    )
    def gather_k(idx_hbm, table2_hbm, out_hbm, idx_v, rows_v, sem):
        wid = lax.axis_index("s") * _NC + lax.axis_index("c")
        base = wid * b_per_w
        pltpu.sync_copy(idx_hbm.at[pl.ds(wid * n_chunks, n_chunks)], idx_v)
        copies = []
        for j in range(n_chunks):
            copies.append(
                pltpu.async_copy(
                    table2_hbm.at[idx_v.at[j]],
                    rows_v.at[pl.ds(j * _CHUNK, _CHUNK)],
                    sem,
                )
            )
        for c in copies:
            c.wait()
        pltpu.sync_copy(rows_v, out_hbm.at[pl.ds(base, b_per_w)])

    return gather_k


def _layernorm(x):
    mu = jnp.mean(x, axis=-1, keepdims=True)
    var = jnp.mean((x - mu) ** 2, axis=-1, keepdims=True)
    return (x - mu) / jnp.sqrt(var + _EPS)


def _gelu(x):
    return x * 0.5 * (1.0 + lax.erf(x * (2.0**-0.5)))


def _mlp_body(e2_ref, p_ref, w1_ref, b1_ref, w2_ref, b2_ref, w3_ref, b3_ref,
              o_ref):
    dim = w1_ref.shape[0]
    lo = e2_ref[:, :dim]
    hi = e2_ref[:, dim:]
    e = lo + (hi - lo) * p_ref[...]
    h = jnp.dot(e, w1_ref[...], preferred_element_type=jnp.float32)
    h = _gelu(_layernorm(h + b1_ref[...]))
    h = jnp.dot(h, w2_ref[...], preferred_element_type=jnp.float32)
    h = _gelu(_layernorm(h + b2_ref[...]))
    h = jnp.dot(h, w3_ref[...], preferred_element_type=jnp.float32)
    o_ref[...] = _gelu(h + b3_ref[...])


def _mlp(e2, p, W1, b1, W2, b2, W3, b3, block=2048):
    batch = e2.shape[0]
    grid = batch // block
    full = lambda shape: pl.BlockSpec(shape, lambda i: (0, 0))
    return pl.pallas_call(
        _mlp_body,
        grid=(grid,),
        in_specs=[
            pl.BlockSpec((block, e2.shape[1]), lambda i: (i, 0)),
            pl.BlockSpec((block, 1), lambda i: (i, 0)),
            full(W1.shape),
            full(b1.shape),
            full(W2.shape),
            full(b2.shape),
            full(W3.shape),
            full(b3.shape),
        ],
        out_specs=pl.BlockSpec((block, W3.shape[1]), lambda i: (i, 0)),
        out_shape=jax.ShapeDtypeStruct((batch, W3.shape[1]), jnp.float32),
    )(e2, p, W1, b1, W2, b2, W3, b3)


def kernel(movie_ids, table, W1, b1, W2, b2, W3, b3):
    batch = movie_ids.shape[0]
    dim = table.shape[1]
    ids = movie_ids.astype(jnp.int32)
    table2 = table.reshape(table.shape[0] // 2, 2 * dim)
    idx2 = (ids >> 1).reshape(-1, _CHUNK)
    parity = (ids & 1).astype(jnp.float32).reshape(batch, 1)
    gather = _make_gather(batch, 2 * dim)
    e2 = gather(idx2, table2)
    return _mlp(
        e2,
        parity,
        W1,
        b1.reshape(1, -1),
        W2,
        b2.reshape(1, -1),
        W3,
        b3.reshape(1, -1),
    )


# speedup vs baseline: 1.7360x; 1.7360x over previous
import functools

import jax
import jax.numpy as jnp
from jax import lax
from jax.experimental import pallas as pl
from jax.experimental.pallas import tpu as pltpu
from jax.experimental.pallas import tpu_sc as plsc

_NC = 2
_NS = 16
_NW = _NC * _NS
_CHUNK = 128

_EPS = 1e-5


_BLK = 4096


def _repack_body(t_ref, eye_ref, o_ref):
    eye = eye_ref[...]
    h = _BLK // 2
    lo = lax.dot_general(
        t_ref[:, :h], eye, (((0,), (0,)), ((), ())),
        preferred_element_type=jnp.float32,
    )
    hi = lax.dot_general(
        t_ref[:, h:], eye, (((0,), (0,)), ((), ())),
        preferred_element_type=jnp.float32,
    )
    o_ref[...] = jnp.concatenate([lo, hi], axis=1)


def _repack(tableT):
    dim, rows = tableT.shape
    grid = (rows + _BLK - 1) // _BLK
    eye = jnp.eye(dim, dtype=jnp.float32)
    return pl.pallas_call(
        _repack_body,
        grid=(grid,),
        in_specs=[
            pl.BlockSpec((dim, _BLK), lambda i: (0, i)),
            pl.BlockSpec((dim, dim), lambda i: (0, 0)),
        ],
        out_specs=pl.BlockSpec((_BLK // 2, 2 * dim), lambda i: (i, 0)),
        out_shape=jax.ShapeDtypeStruct((grid * (_BLK // 2), 2 * dim),
                                       jnp.float32),
    )(tableT, eye)


def _make_gather(batch, dim2):
    b_per_w = batch // _NW
    n_chunks = b_per_w // _CHUNK
    assert b_per_w * _NW == batch and n_chunks * _CHUNK == b_per_w

    mesh = plsc.VectorSubcoreMesh(core_axis_name="c", subcore_axis_name="s")

    @functools.partial(
        pl.kernel,
        mesh=mesh,
        out_type=jax.ShapeDtypeStruct((batch, dim2), jnp.float32),
        scratch_types=[
            pltpu.VMEM((n_chunks, _CHUNK), jnp.int32),
            pltpu.VMEM((b_per_w, dim2), jnp.float32),
            pltpu.SemaphoreType.DMA,
        ],
    )
    def gather_k(idx_hbm, table2_hbm, out_hbm, idx_v, rows_v, sem):
        wid = lax.axis_index("s") * _NC + lax.axis_index("c")
        base = wid * b_per_w
        pltpu.sync_copy(idx_hbm.at[pl.ds(wid * n_chunks, n_chunks)], idx_v)
        copies = []
        for j in range(n_chunks):
            copies.append(
                pltpu.async_copy(
                    table2_hbm.at[idx_v.at[j]],
                    rows_v.at[pl.ds(j * _CHUNK, _CHUNK)],
                    sem,
                )
            )
        for c in copies:
            c.wait()
        pltpu.sync_copy(rows_v, out_hbm.at[pl.ds(base, b_per_w)])

    return gather_k


def _layernorm(x):
    mu = jnp.mean(x, axis=-1, keepdims=True)
    var = jnp.mean((x - mu) ** 2, axis=-1, keepdims=True)
    return (x - mu) / jnp.sqrt(var + _EPS)


def _gelu(x):
    return x * 0.5 * (1.0 + lax.erf(x * (2.0**-0.5)))


def _mlp_body(e2_ref, p_ref, w1_ref, b1_ref, w2_ref, b2_ref, w3_ref, b3_ref,
              o_ref):
    dim = w1_ref.shape[0]
    lo = e2_ref[:, :dim]
    hi = e2_ref[:, dim:]
    e = lo + (hi - lo) * p_ref[...]
    h = jnp.dot(e, w1_ref[...], preferred_element_type=jnp.float32)
    h = _gelu(_layernorm(h + b1_ref[...]))
    h = jnp.dot(h, w2_ref[...], preferred_element_type=jnp.float32)
    h = _gelu(_layernorm(h + b2_ref[...]))
    h = jnp.dot(h, w3_ref[...], preferred_element_type=jnp.float32)
    o_ref[...] = _gelu(h + b3_ref[...])


def _mlp(e2, p, W1, b1, W2, b2, W3, b3, block=2048):
    batch = e2.shape[0]
    grid = batch // block
    full = lambda shape: pl.BlockSpec(shape, lambda i: (0, 0))
    return pl.pallas_call(
        _mlp_body,
        grid=(grid,),
        in_specs=[
            pl.BlockSpec((block, e2.shape[1]), lambda i: (i, 0)),
            pl.BlockSpec((block, 1), lambda i: (i, 0)),
            full(W1.shape),
            full(b1.shape),
            full(W2.shape),
            full(b2.shape),
            full(W3.shape),
            full(b3.shape),
        ],
        out_specs=pl.BlockSpec((block, W3.shape[1]), lambda i: (i, 0)),
        out_shape=jax.ShapeDtypeStruct((batch, W3.shape[1]), jnp.float32),
    )(e2, p, W1, b1, W2, b2, W3, b3)


def kernel(movie_ids, table, W1, b1, W2, b2, W3, b3):
    batch = movie_ids.shape[0]
    ids = movie_ids.astype(jnp.int32)
    table2 = _repack(table.T)
    h = _BLK // 2
    idx2 = (((ids // _BLK) * h) + (ids % h)).reshape(-1, _CHUNK)
    sel = ((ids % _BLK) // h).astype(jnp.float32).reshape(batch, 1)
    gather = _make_gather(batch, table2.shape[1])
    e2 = gather(idx2, table2)
    return _mlp(
        e2,
        sel,
        W1,
        b1.reshape(1, -1),
        W2,
        b2.reshape(1, -1),
        W3,
        b3.reshape(1, -1),
    )


# speedup vs baseline: 2.1082x; 1.2144x over previous
import functools

import jax
import jax.numpy as jnp
from jax import lax
from jax.experimental import pallas as pl
from jax.experimental.pallas import tpu as pltpu
from jax.experimental.pallas import tpu_sc as plsc

_NC = 2
_NS = 16
_NW = _NC * _NS
_CHUNK = 128

_EPS = 1e-5


_BLK = 8192


def _repack_body(t_ref, o_ref):
    h = _BLK // 2
    lo = jnp.swapaxes(t_ref[:, :h], 0, 1)
    hi = jnp.swapaxes(t_ref[:, h:], 0, 1)
    o_ref[...] = jnp.concatenate([lo, hi], axis=1)


def _repack(tableT):
    dim, rows = tableT.shape
    grid = (rows + _BLK - 1) // _BLK
    return pl.pallas_call(
        _repack_body,
        grid=(grid,),
        in_specs=[
            pl.BlockSpec((dim, _BLK), lambda i: (0, i)),
        ],
        out_specs=pl.BlockSpec((_BLK // 2, 2 * dim), lambda i: (i, 0)),
        out_shape=jax.ShapeDtypeStruct((grid * (_BLK // 2), 2 * dim),
                                       jnp.float32),
    )(tableT)


def _make_gather(batch, dim2):
    b_per_w = batch // _NW
    n_chunks = b_per_w // _CHUNK
    assert b_per_w * _NW == batch and n_chunks * _CHUNK == b_per_w

    mesh = plsc.VectorSubcoreMesh(core_axis_name="c", subcore_axis_name="s")

    @functools.partial(
        pl.kernel,
        mesh=mesh,
        out_type=jax.ShapeDtypeStruct((batch, dim2), jnp.float32),
        scratch_types=[
            pltpu.VMEM((n_chunks, _CHUNK), jnp.int32),
            pltpu.VMEM((b_per_w, dim2), jnp.float32),
            pltpu.SemaphoreType.DMA,
        ],
    )
    def gather_k(idx_hbm, table2_hbm, out_hbm, idx_v, rows_v, sem):
        wid = lax.axis_index("s") * _NC + lax.axis_index("c")
        base = wid * b_per_w
        pltpu.sync_copy(idx_hbm.at[pl.ds(wid * n_chunks, n_chunks)], idx_v)
        copies = []
        for j in range(n_chunks):
            copies.append(
                pltpu.async_copy(
                    table2_hbm.at[idx_v.at[j]],
                    rows_v.at[pl.ds(j * _CHUNK, _CHUNK)],
                    sem,
                )
            )
        for c in copies:
            c.wait()
        pltpu.sync_copy(rows_v, out_hbm.at[pl.ds(base, b_per_w)])

    return gather_k


def _layernorm(x):
    mu = jnp.mean(x, axis=-1, keepdims=True)
    var = jnp.mean((x - mu) ** 2, axis=-1, keepdims=True)
    return (x - mu) / jnp.sqrt(var + _EPS)


def _gelu(x):
    return x * 0.5 * (1.0 + lax.erf(x * (2.0**-0.5)))


def _mlp_body(e2_ref, p_ref, w1_ref, b1_ref, w2_ref, b2_ref, w3_ref, b3_ref,
              o_ref):
    dim = w1_ref.shape[0]
    lo = e2_ref[:, :dim]
    hi = e2_ref[:, dim:]
    e = lo + (hi - lo) * p_ref[...]
    h = jnp.dot(e, w1_ref[...], preferred_element_type=jnp.float32)
    h = _gelu(_layernorm(h + b1_ref[...]))
    h = jnp.dot(h, w2_ref[...], preferred_element_type=jnp.float32)
    h = _gelu(_layernorm(h + b2_ref[...]))
    h = jnp.dot(h, w3_ref[...], preferred_element_type=jnp.float32)
    o_ref[...] = _gelu(h + b3_ref[...])


def _mlp(e2, p, W1, b1, W2, b2, W3, b3, block=2048):
    batch = e2.shape[0]
    grid = batch // block
    full = lambda shape: pl.BlockSpec(shape, lambda i: (0, 0))
    return pl.pallas_call(
        _mlp_body,
        grid=(grid,),
        in_specs=[
            pl.BlockSpec((block, e2.shape[1]), lambda i: (i, 0)),
            pl.BlockSpec((block, 1), lambda i: (i, 0)),
            full(W1.shape),
            full(b1.shape),
            full(W2.shape),
            full(b2.shape),
            full(W3.shape),
            full(b3.shape),
        ],
        out_specs=pl.BlockSpec((block, W3.shape[1]), lambda i: (i, 0)),
        out_shape=jax.ShapeDtypeStruct((batch, W3.shape[1]), jnp.float32),
    )(e2, p, W1, b1, W2, b2, W3, b3)


def kernel(movie_ids, table, W1, b1, W2, b2, W3, b3):
    batch = movie_ids.shape[0]
    ids = movie_ids.astype(jnp.int32)
    table2 = _repack(table.T)
    h = _BLK // 2
    idx2 = (((ids // _BLK) * h) + (ids % h)).reshape(-1, _CHUNK)
    sel = ((ids % _BLK) // h).astype(jnp.float32).reshape(batch, 1)
    gather = _make_gather(batch, table2.shape[1])
    e2 = gather(idx2, table2)
    return _mlp(
        e2,
        sel,
        W1,
        b1.reshape(1, -1),
        W2,
        b2.reshape(1, -1),
        W3,
        b3.reshape(1, -1),
    )


# speedup vs baseline: 2.1189x; 1.0051x over previous
import functools

import jax
import jax.numpy as jnp
from jax import lax
from jax.experimental import pallas as pl
from jax.experimental.pallas import tpu as pltpu
from jax.experimental.pallas import tpu_sc as plsc

_NC = 2
_NS = 16
_NW = _NC * _NS
_CHUNK = 128

_EPS = 1e-5


_BLK = 8192


def _repack_body(t_ref, eye_ref, o_ref):
    h = _BLK // 2
    lo = lax.dot_general(
        t_ref[:, :h], eye_ref[...], (((0,), (0,)), ((), ())),
        preferred_element_type=jnp.float32,
    )
    hi = jnp.swapaxes(t_ref[:, h:], 0, 1)
    o_ref[...] = jnp.concatenate([lo, hi], axis=1)


def _repack(tableT):
    dim, rows = tableT.shape
    grid = (rows + _BLK - 1) // _BLK
    eye = jnp.eye(dim, dtype=jnp.float32)
    return pl.pallas_call(
        _repack_body,
        grid=(grid,),
        in_specs=[
            pl.BlockSpec((dim, _BLK), lambda i: (0, i)),
            pl.BlockSpec((dim, dim), lambda i: (0, 0)),
        ],
        out_specs=pl.BlockSpec((_BLK // 2, 2 * dim), lambda i: (i, 0)),
        out_shape=jax.ShapeDtypeStruct((grid * (_BLK // 2), 2 * dim),
                                       jnp.float32),
    )(tableT, eye)


def _make_gather(batch, dim2):
    b_per_w = batch // _NW
    n_chunks = b_per_w // _CHUNK
    assert b_per_w * _NW == batch and n_chunks * _CHUNK == b_per_w

    mesh = plsc.VectorSubcoreMesh(core_axis_name="c", subcore_axis_name="s")

    @functools.partial(
        pl.kernel,
        mesh=mesh,
        out_type=jax.ShapeDtypeStruct((batch, dim2), jnp.float32),
        scratch_types=[
            pltpu.VMEM((n_chunks, _CHUNK), jnp.int32),
            pltpu.VMEM((b_per_w, dim2), jnp.float32),
            pltpu.SemaphoreType.DMA,
        ],
    )
    def gather_k(idx_hbm, table2_hbm, out_hbm, idx_v, rows_v, sem):
        wid = lax.axis_index("s") * _NC + lax.axis_index("c")
        base = wid * b_per_w
        pltpu.sync_copy(idx_hbm.at[pl.ds(wid * n_chunks, n_chunks)], idx_v)
        copies = []
        for j in range(n_chunks):
            copies.append(
                pltpu.async_copy(
                    table2_hbm.at[idx_v.at[j]],
                    rows_v.at[pl.ds(j * _CHUNK, _CHUNK)],
                    sem,
                )
            )
        for c in copies:
            c.wait()
        pltpu.sync_copy(rows_v, out_hbm.at[pl.ds(base, b_per_w)])

    return gather_k


def _layernorm(x):
    mu = jnp.mean(x, axis=-1, keepdims=True)
    var = jnp.mean((x - mu) ** 2, axis=-1, keepdims=True)
    return (x - mu) / jnp.sqrt(var + _EPS)


def _gelu(x):
    return x * 0.5 * (1.0 + lax.erf(x * (2.0**-0.5)))


def _mlp_body(e2_ref, p_ref, w1_ref, b1_ref, w2_ref, b2_ref, w3_ref, b3_ref,
              o_ref):
    dim = w1_ref.shape[0]
    lo = e2_ref[:, :dim]
    hi = e2_ref[:, dim:]
    e = lo + (hi - lo) * p_ref[...]
    h = jnp.dot(e, w1_ref[...], preferred_element_type=jnp.float32)
    h = _gelu(_layernorm(h + b1_ref[...]))
    h = jnp.dot(h, w2_ref[...], preferred_element_type=jnp.float32)
    h = _gelu(_layernorm(h + b2_ref[...]))
    h = jnp.dot(h, w3_ref[...], preferred_element_type=jnp.float32)
    o_ref[...] = _gelu(h + b3_ref[...])


def _mlp(e2, p, W1, b1, W2, b2, W3, b3, block=2048):
    batch = e2.shape[0]
    grid = batch // block
    full = lambda shape: pl.BlockSpec(shape, lambda i: (0, 0))
    return pl.pallas_call(
        _mlp_body,
        grid=(grid,),
        in_specs=[
            pl.BlockSpec((block, e2.shape[1]), lambda i: (i, 0)),
            pl.BlockSpec((block, 1), lambda i: (i, 0)),
            full(W1.shape),
            full(b1.shape),
            full(W2.shape),
            full(b2.shape),
            full(W3.shape),
            full(b3.shape),
        ],
        out_specs=pl.BlockSpec((block, W3.shape[1]), lambda i: (i, 0)),
        out_shape=jax.ShapeDtypeStruct((batch, W3.shape[1]), jnp.float32),
    )(e2, p, W1, b1, W2, b2, W3, b3)


def kernel(movie_ids, table, W1, b1, W2, b2, W3, b3):
    batch = movie_ids.shape[0]
    ids = movie_ids.astype(jnp.int32)
    table2 = _repack(table.T)
    h = _BLK // 2
    idx2 = (((ids // _BLK) * h) + (ids % h)).reshape(-1, _CHUNK)
    sel = ((ids % _BLK) // h).astype(jnp.float32).reshape(batch, 1)
    gather = _make_gather(batch, table2.shape[1])
    e2 = gather(idx2, table2)
    return _mlp(
        e2,
        sel,
        W1,
        b1.reshape(1, -1),
        W2,
        b2.reshape(1, -1),
        W3,
        b3.reshape(1, -1),
    )


# speedup vs baseline: 2.6594x; 1.2550x over previous
import functools

import jax
import jax.numpy as jnp
from jax import lax
from jax.experimental import pallas as pl
from jax.experimental.pallas import tpu as pltpu
from jax.experimental.pallas import tpu_sc as plsc

_NC = 2
_NS = 16
_NW = _NC * _NS
_CHUNK = 128

_EPS = 1e-5


_BLK = 8192


def _repack_body(t_ref, eye_ref, o_ref):
    h = _BLK // 2
    s = jnp.concatenate([t_ref[:, :h], t_ref[:, h:]], axis=0)
    o_ref[...] = lax.dot_general(
        s, eye_ref[...], (((0,), (0,)), ((), ())),
        preferred_element_type=jnp.float32,
    )


def _repack(tableT):
    dim, rows = tableT.shape
    grid = (rows + _BLK - 1) // _BLK
    eye = jnp.eye(2 * dim, dtype=jnp.float32)
    return pl.pallas_call(
        _repack_body,
        grid=(grid,),
        in_specs=[
            pl.BlockSpec((dim, _BLK), lambda i: (0, i)),
            pl.BlockSpec((2 * dim, 2 * dim), lambda i: (0, 0)),
        ],
        out_specs=pl.BlockSpec((_BLK // 2, 2 * dim), lambda i: (i, 0)),
        out_shape=jax.ShapeDtypeStruct((grid * (_BLK // 2), 2 * dim),
                                       jnp.float32),
    )(tableT, eye)


def _make_gather(batch, dim2):
    b_per_w = batch // _NW
    n_chunks = b_per_w // _CHUNK
    assert b_per_w * _NW == batch and n_chunks * _CHUNK == b_per_w

    mesh = plsc.VectorSubcoreMesh(core_axis_name="c", subcore_axis_name="s")

    @functools.partial(
        pl.kernel,
        mesh=mesh,
        out_type=jax.ShapeDtypeStruct((batch, dim2), jnp.float32),
        scratch_types=[
            pltpu.VMEM((n_chunks, _CHUNK), jnp.int32),
            pltpu.VMEM((b_per_w, dim2), jnp.float32),
            pltpu.SemaphoreType.DMA,
        ],
    )
    def gather_k(idx_hbm, table2_hbm, out_hbm, idx_v, rows_v, sem):
        wid = lax.axis_index("s") * _NC + lax.axis_index("c")
        base = wid * b_per_w
        pltpu.sync_copy(idx_hbm.at[pl.ds(wid * n_chunks, n_chunks)], idx_v)
        copies = []
        for j in range(n_chunks):
            copies.append(
                pltpu.async_copy(
                    table2_hbm.at[idx_v.at[j]],
                    rows_v.at[pl.ds(j * _CHUNK, _CHUNK)],
                    sem,
                )
            )
        for c in copies:
            c.wait()
        pltpu.sync_copy(rows_v, out_hbm.at[pl.ds(base, b_per_w)])

    return gather_k


def _layernorm(x):
    mu = jnp.mean(x, axis=-1, keepdims=True)
    var = jnp.mean((x - mu) ** 2, axis=-1, keepdims=True)
    return (x - mu) / jnp.sqrt(var + _EPS)


def _gelu(x):
    return x * 0.5 * (1.0 + lax.erf(x * (2.0**-0.5)))


def _mlp_body(e2_ref, p_ref, w1_ref, b1_ref, w2_ref, b2_ref, w3_ref, b3_ref,
              o_ref):
    dim = w1_ref.shape[0]
    lo = e2_ref[:, :dim]
    hi = e2_ref[:, dim:]
    e = lo + (hi - lo) * p_ref[...]
    h = jnp.dot(e, w1_ref[...], preferred_element_type=jnp.float32)
    h = _gelu(_layernorm(h + b1_ref[...]))
    h = jnp.dot(h, w2_ref[...], preferred_element_type=jnp.float32)
    h = _gelu(_layernorm(h + b2_ref[...]))
    h = jnp.dot(h, w3_ref[...], preferred_element_type=jnp.float32)
    o_ref[...] = _gelu(h + b3_ref[...])


def _mlp(e2, p, W1, b1, W2, b2, W3, b3, block=2048):
    batch = e2.shape[0]
    grid = batch // block
    full = lambda shape: pl.BlockSpec(shape, lambda i: (0, 0))
    return pl.pallas_call(
        _mlp_body,
        grid=(grid,),
        in_specs=[
            pl.BlockSpec((block, e2.shape[1]), lambda i: (i, 0)),
            pl.BlockSpec((block, 1), lambda i: (i, 0)),
            full(W1.shape),
            full(b1.shape),
            full(W2.shape),
            full(b2.shape),
            full(W3.shape),
            full(b3.shape),
        ],
        out_specs=pl.BlockSpec((block, W3.shape[1]), lambda i: (i, 0)),
        out_shape=jax.ShapeDtypeStruct((batch, W3.shape[1]), jnp.float32),
    )(e2, p, W1, b1, W2, b2, W3, b3)


def kernel(movie_ids, table, W1, b1, W2, b2, W3, b3):
    batch = movie_ids.shape[0]
    ids = movie_ids.astype(jnp.int32)
    table2 = _repack(table.T)
    h = _BLK // 2
    idx2 = (((ids // _BLK) * h) + (ids % h)).reshape(-1, _CHUNK)
    sel = ((ids % _BLK) // h).astype(jnp.float32).reshape(batch, 1)
    gather = _make_gather(batch, table2.shape[1])
    e2 = gather(idx2, table2)
    return _mlp(
        e2,
        sel,
        W1,
        b1.reshape(1, -1),
        W2,
        b2.reshape(1, -1),
        W3,
        b3.reshape(1, -1),
    )


# speedup vs baseline: 2.9940x; 1.1258x over previous
import functools

import jax
import jax.numpy as jnp
from jax import lax
from jax.experimental import pallas as pl
from jax.experimental.pallas import tpu as pltpu
from jax.experimental.pallas import tpu_sc as plsc

_NC = 2
_NS = 16
_NW = _NC * _NS
_CHUNK = 128

_EPS = 1e-5


_BLK = 16384


def _repack_body(t_ref, eye_ref, o_ref):
    h = _BLK // 2
    s = jnp.concatenate([t_ref[:, :h], t_ref[:, h:]], axis=0)
    o_ref[...] = lax.dot_general(
        s, eye_ref[...], (((0,), (0,)), ((), ())),
        preferred_element_type=jnp.float32,
    )


def _repack(tableT):
    dim, rows = tableT.shape
    grid = (rows + _BLK - 1) // _BLK
    eye = jnp.eye(2 * dim, dtype=jnp.float32)
    return pl.pallas_call(
        _repack_body,
        grid=(grid,),
        compiler_params=pltpu.CompilerParams(
            vmem_limit_bytes=100 * 1024 * 1024
        ),
        in_specs=[
            pl.BlockSpec((dim, _BLK), lambda i: (0, i)),
            pl.BlockSpec((2 * dim, 2 * dim), lambda i: (0, 0)),
        ],
        out_specs=pl.BlockSpec((_BLK // 2, 2 * dim), lambda i: (i, 0)),
        out_shape=jax.ShapeDtypeStruct((grid * (_BLK // 2), 2 * dim),
                                       jnp.float32),
    )(tableT, eye)


def _make_gather(batch, dim2):
    b_per_w = batch // _NW
    n_chunks = b_per_w // _CHUNK
    assert b_per_w * _NW == batch and n_chunks * _CHUNK == b_per_w

    mesh = plsc.VectorSubcoreMesh(core_axis_name="c", subcore_axis_name="s")

    @functools.partial(
        pl.kernel,
        mesh=mesh,
        out_type=jax.ShapeDtypeStruct((batch, dim2), jnp.float32),
        scratch_types=[
            pltpu.VMEM((n_chunks, _CHUNK), jnp.int32),
            pltpu.VMEM((b_per_w, dim2), jnp.float32),
            pltpu.SemaphoreType.DMA,
        ],
    )
    def gather_k(idx_hbm, table2_hbm, out_hbm, idx_v, rows_v, sem):
        wid = lax.axis_index("s") * _NC + lax.axis_index("c")
        base = wid * b_per_w
        pltpu.sync_copy(idx_hbm.at[pl.ds(wid * n_chunks, n_chunks)], idx_v)
        copies = []
        for j in range(n_chunks):
            copies.append(
                pltpu.async_copy(
                    table2_hbm.at[idx_v.at[j]],
                    rows_v.at[pl.ds(j * _CHUNK, _CHUNK)],
                    sem,
                )
            )
        for c in copies:
            c.wait()
        pltpu.sync_copy(rows_v, out_hbm.at[pl.ds(base, b_per_w)])

    return gather_k


def _layernorm(x):
    mu = jnp.mean(x, axis=-1, keepdims=True)
    var = jnp.mean((x - mu) ** 2, axis=-1, keepdims=True)
    return (x - mu) / jnp.sqrt(var + _EPS)


def _gelu(x):
    return x * 0.5 * (1.0 + lax.erf(x * (2.0**-0.5)))


def _mlp_body(e2_ref, p_ref, w1_ref, b1_ref, w2_ref, b2_ref, w3_ref, b3_ref,
              o_ref):
    dim = w1_ref.shape[0]
    lo = e2_ref[:, :dim]
    hi = e2_ref[:, dim:]
    e = lo + (hi - lo) * p_ref[...]
    h = jnp.dot(e, w1_ref[...], preferred_element_type=jnp.float32)
    h = _gelu(_layernorm(h + b1_ref[...]))
    h = jnp.dot(h, w2_ref[...], preferred_element_type=jnp.float32)
    h = _gelu(_layernorm(h + b2_ref[...]))
    h = jnp.dot(h, w3_ref[...], preferred_element_type=jnp.float32)
    o_ref[...] = _gelu(h + b3_ref[...])


def _mlp(e2, p, W1, b1, W2, b2, W3, b3, block=2048):
    batch = e2.shape[0]
    grid = batch // block
    full = lambda shape: pl.BlockSpec(shape, lambda i: (0, 0))
    return pl.pallas_call(
        _mlp_body,
        grid=(grid,),
        in_specs=[
            pl.BlockSpec((block, e2.shape[1]), lambda i: (i, 0)),
            pl.BlockSpec((block, 1), lambda i: (i, 0)),
            full(W1.shape),
            full(b1.shape),
            full(W2.shape),
            full(b2.shape),
            full(W3.shape),
            full(b3.shape),
        ],
        out_specs=pl.BlockSpec((block, W3.shape[1]), lambda i: (i, 0)),
        out_shape=jax.ShapeDtypeStruct((batch, W3.shape[1]), jnp.float32),
    )(e2, p, W1, b1, W2, b2, W3, b3)


def kernel(movie_ids, table, W1, b1, W2, b2, W3, b3):
    batch = movie_ids.shape[0]
    ids = movie_ids.astype(jnp.int32)
    table2 = _repack(table.T)
    h = _BLK // 2
    idx2 = (((ids // _BLK) * h) + (ids % h)).reshape(-1, _CHUNK)
    sel = ((ids % _BLK) // h).astype(jnp.float32).reshape(batch, 1)
    gather = _make_gather(batch, table2.shape[1])
    e2 = gather(idx2, table2)
    return _mlp(
        e2,
        sel,
        W1,
        b1.reshape(1, -1),
        W2,
        b2.reshape(1, -1),
        W3,
        b3.reshape(1, -1),
    )


# speedup vs baseline: 3.0629x; 1.0230x over previous
import functools

import jax
import jax.numpy as jnp
from jax import lax
from jax.experimental import pallas as pl
from jax.experimental.pallas import tpu as pltpu
from jax.experimental.pallas import tpu_sc as plsc

_NC = 2
_NS = 16
_NW = _NC * _NS
_CHUNK = 128

_EPS = 1e-5


_BLK = 32768


def _repack_body(t_ref, eye_ref, o_ref):
    h = _BLK // 2
    s = jnp.concatenate([t_ref[:, :h], t_ref[:, h:]], axis=0)
    o_ref[...] = lax.dot_general(
        s, eye_ref[...], (((0,), (0,)), ((), ())),
        preferred_element_type=jnp.float32,
    )


def _repack(tableT):
    dim, rows = tableT.shape
    grid = (rows + _BLK - 1) // _BLK
    eye = jnp.eye(2 * dim, dtype=jnp.float32)
    return pl.pallas_call(
        _repack_body,
        grid=(grid,),
        compiler_params=pltpu.CompilerParams(
            vmem_limit_bytes=100 * 1024 * 1024
        ),
        in_specs=[
            pl.BlockSpec((dim, _BLK), lambda i: (0, i)),
            pl.BlockSpec((2 * dim, 2 * dim), lambda i: (0, 0)),
        ],
        out_specs=pl.BlockSpec((_BLK // 2, 2 * dim), lambda i: (i, 0)),
        out_shape=jax.ShapeDtypeStruct((grid * (_BLK // 2), 2 * dim),
                                       jnp.float32),
    )(tableT, eye)


def _make_gather(batch, dim2):
    b_per_w = batch // _NW
    n_chunks = b_per_w // _CHUNK
    assert b_per_w * _NW == batch and n_chunks * _CHUNK == b_per_w

    mesh = plsc.VectorSubcoreMesh(core_axis_name="c", subcore_axis_name="s")

    @functools.partial(
        pl.kernel,
        mesh=mesh,
        out_type=jax.ShapeDtypeStruct((batch, dim2), jnp.float32),
        scratch_types=[
            pltpu.VMEM((n_chunks, _CHUNK), jnp.int32),
            pltpu.VMEM((b_per_w, dim2), jnp.float32),
            pltpu.SemaphoreType.DMA,
        ],
    )
    def gather_k(idx_hbm, table2_hbm, out_hbm, idx_v, rows_v, sem):
        wid = lax.axis_index("s") * _NC + lax.axis_index("c")
        base = wid * b_per_w
        pltpu.sync_copy(idx_hbm.at[pl.ds(wid * n_chunks, n_chunks)], idx_v)
        copies = []
        for j in range(n_chunks):
            copies.append(
                pltpu.async_copy(
                    table2_hbm.at[idx_v.at[j]],
                    rows_v.at[pl.ds(j * _CHUNK, _CHUNK)],
                    sem,
                )
            )
        for c in copies:
            c.wait()
        pltpu.sync_copy(rows_v, out_hbm.at[pl.ds(base, b_per_w)])

    return gather_k


def _layernorm(x):
    mu = jnp.mean(x, axis=-1, keepdims=True)
    var = jnp.mean((x - mu) ** 2, axis=-1, keepdims=True)
    return (x - mu) / jnp.sqrt(var + _EPS)


def _gelu(x):
    return x * 0.5 * (1.0 + lax.erf(x * (2.0**-0.5)))


def _mlp_body(e2_ref, p_ref, w1_ref, b1_ref, w2_ref, b2_ref, w3_ref, b3_ref,
              o_ref):
    dim = w1_ref.shape[0]
    lo = e2_ref[:, :dim]
    hi = e2_ref[:, dim:]
    e = lo + (hi - lo) * p_ref[...]
    h = jnp.dot(e, w1_ref[...], preferred_element_type=jnp.float32)
    h = _gelu(_layernorm(h + b1_ref[...]))
    h = jnp.dot(h, w2_ref[...], preferred_element_type=jnp.float32)
    h = _gelu(_layernorm(h + b2_ref[...]))
    h = jnp.dot(h, w3_ref[...], preferred_element_type=jnp.float32)
    o_ref[...] = _gelu(h + b3_ref[...])


def _mlp(e2, p, W1, b1, W2, b2, W3, b3, block=2048):
    batch = e2.shape[0]
    grid = batch // block
    full = lambda shape: pl.BlockSpec(shape, lambda i: (0, 0))
    return pl.pallas_call(
        _mlp_body,
        grid=(grid,),
        in_specs=[
            pl.BlockSpec((block, e2.shape[1]), lambda i: (i, 0)),
            pl.BlockSpec((block, 1), lambda i: (i, 0)),
            full(W1.shape),
            full(b1.shape),
            full(W2.shape),
            full(b2.shape),
            full(W3.shape),
            full(b3.shape),
        ],
        out_specs=pl.BlockSpec((block, W3.shape[1]), lambda i: (i, 0)),
        out_shape=jax.ShapeDtypeStruct((batch, W3.shape[1]), jnp.float32),
    )(e2, p, W1, b1, W2, b2, W3, b3)


def kernel(movie_ids, table, W1, b1, W2, b2, W3, b3):
    batch = movie_ids.shape[0]
    ids = movie_ids.astype(jnp.int32)
    table2 = _repack(table.T)
    h = _BLK // 2
    idx2 = (((ids // _BLK) * h) + (ids % h)).reshape(-1, _CHUNK)
    sel = ((ids % _BLK) // h).astype(jnp.float32).reshape(batch, 1)
    gather = _make_gather(batch, table2.shape[1])
    e2 = gather(idx2, table2)
    return _mlp(
        e2,
        sel,
        W1,
        b1.reshape(1, -1),
        W2,
        b2.reshape(1, -1),
        W3,
        b3.reshape(1, -1),
    )
